# baseline (speedup 1.0000x reference)
import functools

import jax
import jax.numpy as jnp
from jax.experimental import pallas as pl
from jax.experimental.pallas import tpu as pltpu

_BLK = 10240


def _fused_gru_kernel(x_ref, wz_ref, bz_ref, wh_ref, bh_ref, wl_ref, out_ref):
    x = x_ref[...]
    t = jnp.tanh(
        jnp.dot(x, wz_ref[...] * 0.5, preferred_element_type=jnp.float32)
        + bz_ref[...] * 0.5)
    ht = jnp.tanh(
        jnp.dot(x, wh_ref[...], preferred_element_type=jnp.float32)
        + bh_ref[...])
    h = (1.0 - t) * jax.nn.relu(ht)
    out_ref[...] = jax.lax.dot_general(
        wl_ref[...] * 0.5, h, (((0,), (1,)), ((), ())),
        preferred_element_type=jnp.float32)


@functools.partial(jax.jit, static_argnames=())
def kernel(x, edge_index, edge_weight, W_xz, b_xz, W_hz, b_hz, W_xr, b_xr,
           W_hr, b_hr, W_xh, b_xh, W_hh, b_hh, W_lin, b_lin):
    n, f_in = x.shape
    out_len = W_lin.shape[1]
    bz = (b_xz + b_hz).reshape(1, -1)
    bh = (b_xh + b_hh).reshape(1, -1)

    steps = pl.cdiv(n, _BLK)
    n_pad = steps * _BLK
    out_t = pl.pallas_call(
        _fused_gru_kernel,
        grid=(steps,),
        in_specs=[
            pl.BlockSpec((_BLK, f_in), lambda i: (i, 0)),
            pl.BlockSpec((f_in, W_xz.shape[1]), lambda i: (0, 0)),
            pl.BlockSpec((1, W_xz.shape[1]), lambda i: (0, 0)),
            pl.BlockSpec((f_in, W_xh.shape[1]), lambda i: (0, 0)),
            pl.BlockSpec((1, W_xh.shape[1]), lambda i: (0, 0)),
            pl.BlockSpec((W_lin.shape[0], out_len), lambda i: (0, 0)),
        ],
        out_specs=pl.BlockSpec((out_len, _BLK), lambda i: (0, i)),
        out_shape=jax.ShapeDtypeStruct((out_len, n_pad), x.dtype),
        compiler_params=pltpu.CompilerParams(
            dimension_semantics=("parallel",)),
    )(x, W_xz, bz, W_xh, bh, W_lin)
    return (out_t[:, :n].T + b_lin[None, :],)


# speedup vs baseline: 1.0656x; 1.0656x over previous
import functools

import jax
import jax.numpy as jnp
from jax.experimental import pallas as pl
from jax.experimental.pallas import tpu as pltpu

_BLK = 5120


def _fused_gru_kernel(x_ref, wz_ref, bz_ref, wh_ref, bh_ref, wl_ref, out_ref):
    x = x_ref[...]
    t = jnp.tanh(
        jnp.dot(x, wz_ref[...] * 0.5, preferred_element_type=jnp.float32)
        + bz_ref[...] * 0.5)
    ht = jnp.tanh(
        jnp.dot(x, wh_ref[...], preferred_element_type=jnp.float32)
        + bh_ref[...])
    h = (1.0 - t) * jax.nn.relu(ht)
    out_ref[...] = jax.lax.dot_general(
        wl_ref[...] * 0.5, h, (((0,), (1,)), ((), ())),
        preferred_element_type=jnp.float32)


@functools.partial(jax.jit, static_argnames=())
def kernel(x, edge_index, edge_weight, W_xz, b_xz, W_hz, b_hz, W_xr, b_xr,
           W_hr, b_hr, W_xh, b_xh, W_hh, b_hh, W_lin, b_lin):
    n, f_in = x.shape
    out_len = W_lin.shape[1]
    bz = (b_xz + b_hz).reshape(1, -1)
    bh = (b_xh + b_hh).reshape(1, -1)

    steps = pl.cdiv(n, _BLK)
    n_pad = steps * _BLK
    out_t = pl.pallas_call(
        _fused_gru_kernel,
        grid=(steps,),
        in_specs=[
            pl.BlockSpec((_BLK, f_in), lambda i: (i, 0)),
            pl.BlockSpec((f_in, W_xz.shape[1]), lambda i: (0, 0)),
            pl.BlockSpec((1, W_xz.shape[1]), lambda i: (0, 0)),
            pl.BlockSpec((f_in, W_xh.shape[1]), lambda i: (0, 0)),
            pl.BlockSpec((1, W_xh.shape[1]), lambda i: (0, 0)),
            pl.BlockSpec((W_lin.shape[0], out_len), lambda i: (0, 0)),
        ],
        out_specs=pl.BlockSpec((out_len, _BLK), lambda i: (0, i)),
        out_shape=jax.ShapeDtypeStruct((out_len, n_pad), x.dtype),
        compiler_params=pltpu.CompilerParams(
            dimension_semantics=("arbitrary",)),
    )(x, W_xz, bz, W_xh, bh, W_lin)
    return (out_t[:, :n].T + b_lin[None, :],)
